# Initial kernel scaffold; baseline (speedup 1.0000x reference)
import jax
import jax.numpy as jnp
from jax.experimental import pallas as pl


def kernel(x_chemical, x_gene, edge_index_cg, edge_index_gc, edge_attr_cg, edge_attr_gc, params):
    raise NotImplementedError("write your pallas kernel here")


# speedup vs baseline: 1.9505x; 1.9505x over previous
import jax
import jax.numpy as jnp
from jax.experimental import pallas as pl

_C = 128
_H = 4
_D = 32


def _mm_bias(x, W, b, sigmoid=False, block_rows=None):
    n = x.shape[0]
    if block_rows is None:
        block_rows = n
    grid = (n // block_rows,)

    def body(x_ref, w_ref, b_ref, o_ref):
        acc = jnp.dot(x_ref[...], w_ref[...], preferred_element_type=jnp.float32)
        acc = acc + b_ref[...]
        if sigmoid:
            acc = jax.nn.sigmoid(acc)
        o_ref[...] = acc

    return pl.pallas_call(
        body,
        grid=grid,
        in_specs=[
            pl.BlockSpec((block_rows, x.shape[1]), lambda i: (i, 0)),
            pl.BlockSpec((x.shape[1], W.shape[1]), lambda i: (0, 0)),
            pl.BlockSpec((1, W.shape[1]), lambda i: (0, 0)),
        ],
        out_specs=pl.BlockSpec((block_rows, W.shape[1]), lambda i: (i, 0)),
        out_shape=jax.ShapeDtypeStruct((n, W.shape[1]), jnp.float32),
    )(x, W, b.reshape(1, -1))


def _final(agg, s, W_out, b_out, S4, block_rows=1000):
    n = agg.shape[0]

    def body(a_ref, s_ref, w_ref, b_ref, s4_ref, o_ref):
        srep = jnp.dot(s_ref[...], s4_ref[...], preferred_element_type=jnp.float32)
        norm = a_ref[...] / (srep + 1e-16)
        o_ref[...] = jnp.dot(norm, w_ref[...], preferred_element_type=jnp.float32) + b_ref[...]

    return pl.pallas_call(
        body,
        grid=(n // block_rows,),
        in_specs=[
            pl.BlockSpec((block_rows, _C), lambda i: (i, 0)),
            pl.BlockSpec((block_rows, _H), lambda i: (i, 0)),
            pl.BlockSpec((_C, _C), lambda i: (0, 0)),
            pl.BlockSpec((1, _C), lambda i: (0, 0)),
            pl.BlockSpec((_H, _C), lambda i: (0, 0)),
        ],
        out_specs=pl.BlockSpec((block_rows, _C), lambda i: (i, 0)),
        out_shape=jax.ShapeDtypeStruct((n, _C), jnp.float32),
    )(agg, s, W_out, b_out.reshape(1, -1), S4)


def _conv(x_src, x_dst, ei, ea, p, ek, emb_t, emb_s, n_dst):
    src, dst = ei[0], ei[1]
    n_e = ei.shape[1]

    Hs = _mm_bias(x_src, p["W_src_" + ek], p["b_src_" + ek], block_rows=1000)
    Hd = _mm_bias(x_dst, p["W_dst_" + ek], p["b_dst_" + ek], block_rows=1000)

    n_t, n_s = emb_t.shape[0], emb_s.shape[0]
    cat = jnp.concatenate(
        [jnp.repeat(emb_t, n_s, axis=0), jnp.tile(emb_s, (n_t, 1))], axis=1
    )
    G = _mm_bias(cat, p["W_gate_" + ek], p["b_gate_" + ek], sigmoid=True)

    attn = p["attn_" + ek].reshape(-1)
    head_of = jnp.arange(_C) // _D
    A = jnp.zeros((_C, _H), jnp.float32).at[jnp.arange(_C), head_of].set(attn)
    S4 = (head_of[None, :] == jnp.arange(_H)[:, None]).astype(jnp.float32)

    combo = ea[:, 0] * n_s + ea[:, 1]
    hs = jnp.take(Hs, src, axis=0)
    hd = jnp.take(Hd, dst, axis=0)
    gate = jnp.take(G, combo, axis=0)
    t = hs + hd
    t = jnp.where(t >= 0, t, 0.2 * t)
    logits = t @ A
    ex = jnp.exp(jnp.minimum(logits, 60.0))
    pw = hs * gate * (ex @ S4)
    agg = jnp.zeros((n_dst, _C), jnp.float32).at[dst].add(pw)
    s = jnp.zeros((n_dst, _H), jnp.float32).at[dst].add(ex)
    return agg, s, S4


def kernel(x_chemical, x_gene, edge_index_cg, edge_index_gc, edge_attr_cg, edge_attr_gc, params):
    p = params
    emb_t, emb_s = p["emb_action_type"], p["emb_action_subject"]
    agg_g, s_g, S4 = _conv(x_chemical, x_gene, edge_index_cg, edge_attr_cg, p, "cg",
                           emb_t, emb_s, x_gene.shape[0])
    agg_c, s_c, _ = _conv(x_gene, x_chemical, edge_index_gc, edge_attr_gc, p, "gc",
                          emb_t, emb_s, x_chemical.shape[0])
    out_chemical = _final(agg_c, s_c, p["W_out_chemical"], p["b_out_chemical"], S4)
    out_gene = _final(agg_g, s_g, p["W_out_gene"], p["b_out_gene"], S4)
    return (out_chemical, out_gene)


# speedup vs baseline: 4.7090x; 2.4143x over previous
import functools

import jax
import jax.numpy as jnp
from jax import lax
from jax.experimental import pallas as pl
from jax.experimental.pallas import tpu as pltpu
from jax.experimental.pallas import tpu_sc as plsc

_C = 128
_H = 4
_D = 32
_NTILE = 32
_CHUNK = 128
_EBLK = 1280


def _mm_bias(x, W, b, sigmoid=False, block_rows=None):
    n = x.shape[0]
    if block_rows is None:
        block_rows = n

    def body(x_ref, w_ref, b_ref, o_ref):
        acc = jnp.dot(x_ref[...], w_ref[...], preferred_element_type=jnp.float32)
        acc = acc + b_ref[...]
        if sigmoid:
            acc = jax.nn.sigmoid(acc)
        o_ref[...] = acc

    return pl.pallas_call(
        body,
        grid=(n // block_rows,),
        in_specs=[
            pl.BlockSpec((block_rows, x.shape[1]), lambda i: (i, 0)),
            pl.BlockSpec((x.shape[1], W.shape[1]), lambda i: (0, 0)),
            pl.BlockSpec((1, W.shape[1]), lambda i: (0, 0)),
        ],
        out_specs=pl.BlockSpec((block_rows, W.shape[1]), lambda i: (i, 0)),
        out_shape=jax.ShapeDtypeStruct((n, W.shape[1]), jnp.float32),
    )(x, W, b.reshape(1, -1))


def _sc_gather(Hs, Hd, src, dst):
    E = src.shape[0]
    per = E // _NTILE
    nfull = per // _CHUNK
    tail = per - nfull * _CHUNK
    mesh = plsc.VectorSubcoreMesh(core_axis_name="c", subcore_axis_name="s")

    @functools.partial(
        pl.kernel,
        out_type=[jax.ShapeDtypeStruct((E, _C), jnp.float32),
                  jax.ShapeDtypeStruct((E, _C), jnp.float32)],
        mesh=mesh,
        scratch_types=[
            pltpu.VMEM((_CHUNK,), jnp.int32),
            pltpu.VMEM((_CHUNK,), jnp.int32),
            pltpu.VMEM((_CHUNK, _C), jnp.float32),
            pltpu.VMEM((_CHUNK, _C), jnp.float32),
            pltpu.VMEM((max(tail, 8),), jnp.int32),
            pltpu.VMEM((max(tail, 8),), jnp.int32),
            pltpu.VMEM((max(tail, 8), _C), jnp.float32),
            pltpu.VMEM((max(tail, 8), _C), jnp.float32),
            pltpu.SemaphoreType.DMA,
            pltpu.SemaphoreType.DMA,
        ],
    )
    def k(hs_hbm, hd_hbm, src_hbm, dst_hbm, os_hbm, od_hbm,
          isv, idv, rs, rd, isvt, idvt, rst, rdt, sem1, sem2):
        wid = lax.axis_index("s") * 2 + lax.axis_index("c")
        base = wid * per

        def do_chunk(off, n, iref_s, iref_d, rref_s, rref_d):
            pltpu.sync_copy(src_hbm.at[pl.ds(off, n)], iref_s)
            pltpu.sync_copy(dst_hbm.at[pl.ds(off, n)], iref_d)
            c1 = pltpu.async_copy(hs_hbm.at[iref_s], rref_s, sem1)
            c2 = pltpu.async_copy(hd_hbm.at[iref_d], rref_d, sem2)
            c1.wait()
            c2.wait()
            pltpu.sync_copy(rref_s, os_hbm.at[pl.ds(off, n)])
            pltpu.sync_copy(rref_d, od_hbm.at[pl.ds(off, n)])

        @pl.loop(0, nfull)
        def _(i):
            do_chunk(base + i * _CHUNK, _CHUNK, isv, idv, rs, rd)

        if tail:
            do_chunk(base + nfull * _CHUNK, tail, isvt, idvt, rst, rdt)

    return k(Hs, Hd, src, dst)


def _sc_scatter(pw, dst, zc):
    E = pw.shape[0]
    per = E // _NTILE
    nfull = per // _CHUNK
    tail = per - nfull * _CHUNK
    N = zc.shape[0]
    stripe = (N // 16) & ~7
    rem = N - stripe * 16
    mesh = plsc.VectorSubcoreMesh(core_axis_name="c", subcore_axis_name="s")

    @functools.partial(
        pl.kernel,
        out_type=jax.ShapeDtypeStruct((2, N, _C), jnp.float32),
        mesh=mesh,
        scratch_types=[
            pltpu.VMEM((_CHUNK,), jnp.int32),
            pltpu.VMEM((max(tail, 8),), jnp.int32),
            pltpu.VMEM((_CHUNK, _C), jnp.float32),
            pltpu.VMEM((max(tail, 8), _C), jnp.float32),
            pltpu.VMEM((16, _C), jnp.float32),
            pltpu.VMEM_SHARED((N, _C), jnp.float32),
        ],
    )
    def k(pw_hbm, dst_hbm, zc_hbm, oa_hbm,
          idx, idxt, pv, pvt, stg, aggsh):
        cid = lax.axis_index("c")
        sid = lax.axis_index("s")
        wid = sid * 2 + cid
        base = wid * per
        r0 = sid * stripe
        nz = stripe // 16

        pltpu.sync_copy(zc_hbm.at[pl.ds(0, 16)], stg)

        @pl.loop(0, nz)
        def _(i):
            pltpu.sync_copy(stg, aggsh.at[pl.ds(r0 + i * 16, 16)])

        @pl.when(sid == 15)
        def _():
            @pl.loop(0, rem // 16)
            def _(i):
                pltpu.sync_copy(stg, aggsh.at[pl.ds(stripe * 16 + i * 16, 16)])

        plsc.subcore_barrier()

        @pl.loop(0, nfull)
        def _(i):
            off = base + i * _CHUNK
            pltpu.sync_copy(dst_hbm.at[pl.ds(off, _CHUNK)], idx)
            pltpu.sync_copy(pw_hbm.at[pl.ds(off, _CHUNK)], pv)
            pltpu.sync_copy(pv, aggsh.at[idx], add=True)

        if tail:
            off = base + nfull * _CHUNK
            pltpu.sync_copy(dst_hbm.at[pl.ds(off, tail)], idxt)
            pltpu.sync_copy(pw_hbm.at[pl.ds(off, tail)], pvt)
            pltpu.sync_copy(pvt, aggsh.at[idxt], add=True)

        plsc.subcore_barrier()

        @pl.loop(0, nz)
        def _(i):
            pltpu.sync_copy(aggsh.at[pl.ds(r0 + i * 16, 16)], stg)
            pltpu.sync_copy(stg, oa_hbm.at[cid].at[pl.ds(r0 + i * 16, 16)])

        @pl.when(sid == 15)
        def _():
            @pl.loop(0, rem // 16)
            def _(i):
                pltpu.sync_copy(aggsh.at[pl.ds(stripe * 16 + i * 16, 16)], stg)
                pltpu.sync_copy(stg, oa_hbm.at[cid].at[pl.ds(stripe * 16 + i * 16, 16)])

    return k(pw, dst, zc)


def _tc_middle(hs_e, hd_e, combo3, G, A, S4):
    E = hs_e.shape[0]
    nblk = E // _EBLK

    def body(hs_ref, hd_ref, c_ref, g_ref, a_ref, s4_ref, opw_ref, oex_ref):
        hs = hs_ref[...]
        t = hs + hd_ref[...]
        t = jnp.where(t >= 0, t, 0.2 * t)
        logits = jnp.dot(t, a_ref[...], preferred_element_type=jnp.float32)
        ex = jnp.exp(jnp.minimum(logits, 60.0))
        combo = c_ref[0, 0, :]
        onehot = (combo[:, None]
                  == lax.broadcasted_iota(jnp.int32, (_EBLK, 128), 1)
                  ).astype(jnp.float32)
        gate = jnp.dot(onehot, g_ref[...], preferred_element_type=jnp.float32)
        exb = jnp.dot(ex, s4_ref[...], preferred_element_type=jnp.float32)
        opw_ref[...] = hs * gate * exb
        oex_ref[...] = jnp.concatenate(
            [ex, jnp.zeros((_EBLK, 12), jnp.float32)], axis=1)

    return pl.pallas_call(
        body,
        grid=(nblk,),
        in_specs=[
            pl.BlockSpec((_EBLK, _C), lambda i: (i, 0)),
            pl.BlockSpec((_EBLK, _C), lambda i: (i, 0)),
            pl.BlockSpec((1, 1, _EBLK), lambda i: (i, 0, 0)),
            pl.BlockSpec((128, _C), lambda i: (0, 0)),
            pl.BlockSpec((_C, _H), lambda i: (0, 0)),
            pl.BlockSpec((_H, _C), lambda i: (0, 0)),
        ],
        out_specs=[
            pl.BlockSpec((_EBLK, _C), lambda i: (i, 0)),
            pl.BlockSpec((_EBLK, 16), lambda i: (i, 0)),
        ],
        out_shape=[
            jax.ShapeDtypeStruct((E, _C), jnp.float32),
            jax.ShapeDtypeStruct((E, 16), jnp.float32),
        ],
    )(hs_e, hd_e, combo3, G, A, S4)


def _final(agg_p, s_p, W_out, b_out, S16, block_rows=1000):
    n = agg_p.shape[1]

    def body(a_ref, s_ref, w_ref, b_ref, s16_ref, o_ref):
        agg = a_ref[0] + a_ref[1]
        s = s_ref[0] + s_ref[1]
        srep = jnp.dot(s, s16_ref[...], preferred_element_type=jnp.float32)
        norm = agg / (srep + 1e-16)
        o_ref[...] = jnp.dot(norm, w_ref[...],
                             preferred_element_type=jnp.float32) + b_ref[...]

    return pl.pallas_call(
        body,
        grid=(n // block_rows,),
        in_specs=[
            pl.BlockSpec((2, block_rows, _C), lambda i: (0, i, 0)),
            pl.BlockSpec((2, block_rows, 16), lambda i: (0, i, 0)),
            pl.BlockSpec((_C, _C), lambda i: (0, 0)),
            pl.BlockSpec((1, _C), lambda i: (0, 0)),
            pl.BlockSpec((16, _C), lambda i: (0, 0)),
        ],
        out_specs=pl.BlockSpec((block_rows, _C), lambda i: (i, 0)),
        out_shape=jax.ShapeDtypeStruct((n, _C), jnp.float32),
    )(agg_p, s_p, W_out, b_out.reshape(1, -1), S16)


def _conv(x_src, x_dst, ei, ea, p, ek, emb_t, emb_s, n_dst):
    src, dst = ei[0], ei[1]
    E = ei.shape[1]

    Hs = _mm_bias(x_src, p["W_src_" + ek], p["b_src_" + ek], block_rows=1000)
    Hd = _mm_bias(x_dst, p["W_dst_" + ek], p["b_dst_" + ek], block_rows=1000)

    n_t, n_s = emb_t.shape[0], emb_s.shape[0]
    cat = jnp.concatenate(
        [jnp.repeat(emb_t, n_s, axis=0), jnp.tile(emb_s, (n_t, 1))], axis=1)
    G = _mm_bias(cat, p["W_gate_" + ek], p["b_gate_" + ek], sigmoid=True)

    attn = p["attn_" + ek].reshape(-1)
    head_of = jnp.arange(_C) // _D
    A = jnp.zeros((_C, _H), jnp.float32).at[jnp.arange(_C), head_of].set(attn)
    S4 = (head_of[None, :] == jnp.arange(_H)[:, None]).astype(jnp.float32)
    S16 = jnp.zeros((16, _C), jnp.float32).at[:_H].set(S4)

    combo3 = (ea[:, 0] * n_s + ea[:, 1]).astype(jnp.int32).reshape(
        E // _EBLK, 1, _EBLK)

    hs_e, hd_e = _sc_gather(Hs, Hd, src, dst)
    pw, exs = _tc_middle(hs_e, hd_e, combo3, G, A, S4)
    zc = jnp.zeros((n_dst, _C), jnp.float32)
    agg_p = _sc_scatter(pw, dst, zc)
    s = jnp.zeros((n_dst, 16), jnp.float32).at[dst].add(exs)
    s_p = jnp.stack([s, jnp.zeros_like(s)])
    return agg_p, s_p, S16


def kernel(x_chemical, x_gene, edge_index_cg, edge_index_gc, edge_attr_cg,
           edge_attr_gc, params):
    p = params
    emb_t, emb_s = p["emb_action_type"], p["emb_action_subject"]
    agg_g, s_g, S16 = _conv(x_chemical, x_gene, edge_index_cg, edge_attr_cg,
                            p, "cg", emb_t, emb_s, x_gene.shape[0])
    agg_c, s_c, _ = _conv(x_gene, x_chemical, edge_index_gc, edge_attr_gc,
                          p, "gc", emb_t, emb_s, x_chemical.shape[0])
    out_chemical = _final(agg_c, s_c, p["W_out_chemical"], p["b_out_chemical"], S16)
    out_gene = _final(agg_g, s_g, p["W_out_gene"], p["b_out_gene"], S16)
    return (out_chemical, out_gene)


# speedup vs baseline: 7.0147x; 1.4896x over previous
import functools

import jax
import jax.numpy as jnp
from jax import lax
from jax.experimental import pallas as pl
from jax.experimental.pallas import tpu as pltpu
from jax.experimental.pallas import tpu_sc as plsc

_C = 128
_H = 4
_D = 32
_NTILE = 32
_CHUNK = 128
_EBLK = 1280


def _mm_bias(x, W, b, sigmoid=False, block_rows=None):
    n = x.shape[0]
    if block_rows is None:
        block_rows = n

    def body(x_ref, w_ref, b_ref, o_ref):
        acc = jnp.dot(x_ref[...], w_ref[...], preferred_element_type=jnp.float32)
        acc = acc + b_ref[...]
        if sigmoid:
            acc = jax.nn.sigmoid(acc)
        o_ref[...] = acc

    return pl.pallas_call(
        body,
        grid=(n // block_rows,),
        in_specs=[
            pl.BlockSpec((block_rows, x.shape[1]), lambda i: (i, 0)),
            pl.BlockSpec((x.shape[1], W.shape[1]), lambda i: (0, 0)),
            pl.BlockSpec((1, W.shape[1]), lambda i: (0, 0)),
        ],
        out_specs=pl.BlockSpec((block_rows, W.shape[1]), lambda i: (i, 0)),
        out_shape=jax.ShapeDtypeStruct((n, W.shape[1]), jnp.float32),
    )(x, W, b.reshape(1, -1))


def _sc_gather(Hs, Hd, src, dst):
    E = src.shape[0]
    per = E // _NTILE
    nfull = per // _CHUNK
    tail = per - nfull * _CHUNK
    mesh = plsc.VectorSubcoreMesh(core_axis_name="c", subcore_axis_name="s")

    @functools.partial(
        pl.kernel,
        out_type=[jax.ShapeDtypeStruct((E, _C), jnp.float32),
                  jax.ShapeDtypeStruct((E, _C), jnp.float32)],
        mesh=mesh,
        scratch_types=[
            pltpu.VMEM((_CHUNK,), jnp.int32),
            pltpu.VMEM((_CHUNK,), jnp.int32),
            pltpu.VMEM((_CHUNK, _C), jnp.float32),
            pltpu.VMEM((_CHUNK, _C), jnp.float32),
            pltpu.VMEM((max(tail, 8),), jnp.int32),
            pltpu.VMEM((max(tail, 8),), jnp.int32),
            pltpu.VMEM((max(tail, 8), _C), jnp.float32),
            pltpu.VMEM((max(tail, 8), _C), jnp.float32),
            pltpu.SemaphoreType.DMA,
            pltpu.SemaphoreType.DMA,
        ],
    )
    def k(hs_hbm, hd_hbm, src_hbm, dst_hbm, os_hbm, od_hbm,
          isv, idv, rs, rd, isvt, idvt, rst, rdt, sem1, sem2):
        wid = lax.axis_index("s") * 2 + lax.axis_index("c")
        base = wid * per

        def do_chunk(off, n, iref_s, iref_d, rref_s, rref_d):
            pltpu.sync_copy(src_hbm.at[pl.ds(off, n)], iref_s)
            pltpu.sync_copy(dst_hbm.at[pl.ds(off, n)], iref_d)
            c1 = pltpu.async_copy(hs_hbm.at[iref_s], rref_s, sem1)
            c2 = pltpu.async_copy(hd_hbm.at[iref_d], rref_d, sem2)
            c1.wait()
            c2.wait()
            pltpu.sync_copy(rref_s, os_hbm.at[pl.ds(off, n)])
            pltpu.sync_copy(rref_d, od_hbm.at[pl.ds(off, n)])

        @pl.loop(0, nfull)
        def _(i):
            do_chunk(base + i * _CHUNK, _CHUNK, isv, idv, rs, rd)

        if tail:
            do_chunk(base + nfull * _CHUNK, tail, isvt, idvt, rst, rdt)

    return k(Hs, Hd, src, dst)


def _sc_scatter(pw, dst, zc):
    E = pw.shape[0]
    per = E // _NTILE
    nfull = per // _CHUNK
    tail = per - nfull * _CHUNK
    N = zc.shape[0]
    stripe = (N // 16) & ~7
    rem = N - stripe * 16
    mesh = plsc.VectorSubcoreMesh(core_axis_name="c", subcore_axis_name="s")

    @functools.partial(
        pl.kernel,
        out_type=jax.ShapeDtypeStruct((2, N, _C), jnp.float32),
        mesh=mesh,
        scratch_types=[
            pltpu.VMEM((_CHUNK,), jnp.int32),
            pltpu.VMEM((max(tail, 8),), jnp.int32),
            pltpu.VMEM((_CHUNK, _C), jnp.float32),
            pltpu.VMEM((max(tail, 8), _C), jnp.float32),
            pltpu.VMEM((16, _C), jnp.float32),
            pltpu.VMEM_SHARED((N, _C), jnp.float32),
        ],
    )
    def k(pw_hbm, dst_hbm, zc_hbm, oa_hbm,
          idx, idxt, pv, pvt, stg, aggsh):
        cid = lax.axis_index("c")
        sid = lax.axis_index("s")
        wid = sid * 2 + cid
        base = wid * per
        r0 = sid * stripe
        nz = stripe // 16

        pltpu.sync_copy(zc_hbm.at[pl.ds(0, 16)], stg)

        @pl.loop(0, nz)
        def _(i):
            pltpu.sync_copy(stg, aggsh.at[pl.ds(r0 + i * 16, 16)])

        @pl.when(sid == 15)
        def _():
            @pl.loop(0, rem // 16)
            def _(i):
                pltpu.sync_copy(stg, aggsh.at[pl.ds(stripe * 16 + i * 16, 16)])

        plsc.subcore_barrier()

        @pl.loop(0, nfull)
        def _(i):
            off = base + i * _CHUNK
            pltpu.sync_copy(dst_hbm.at[pl.ds(off, _CHUNK)], idx)
            pltpu.sync_copy(pw_hbm.at[pl.ds(off, _CHUNK)], pv)
            pltpu.sync_copy(pv, aggsh.at[idx], add=True)

        if tail:
            off = base + nfull * _CHUNK
            pltpu.sync_copy(dst_hbm.at[pl.ds(off, tail)], idxt)
            pltpu.sync_copy(pw_hbm.at[pl.ds(off, tail)], pvt)
            pltpu.sync_copy(pvt, aggsh.at[idxt], add=True)

        plsc.subcore_barrier()

        @pl.loop(0, nz)
        def _(i):
            pltpu.sync_copy(aggsh.at[pl.ds(r0 + i * 16, 16)], stg)
            pltpu.sync_copy(stg, oa_hbm.at[cid].at[pl.ds(r0 + i * 16, 16)])

        @pl.when(sid == 15)
        def _():
            @pl.loop(0, rem // 16)
            def _(i):
                pltpu.sync_copy(aggsh.at[pl.ds(stripe * 16 + i * 16, 16)], stg)
                pltpu.sync_copy(stg, oa_hbm.at[cid].at[pl.ds(stripe * 16 + i * 16, 16)])

    return k(pw, dst, zc)


def _tc_middle(hs_e, hd_e, combo3, G, A, S4):
    E = hs_e.shape[0]
    nblk = E // _EBLK

    def body(hs_ref, hd_ref, c_ref, g_ref, a_ref, s4_ref, opw_ref, oex_ref):
        hs = hs_ref[...]
        t = hs + hd_ref[...]
        t = jnp.where(t >= 0, t, 0.2 * t)
        logits = jnp.dot(t, a_ref[...], preferred_element_type=jnp.float32)
        ex = jnp.exp(jnp.minimum(logits, 60.0))
        combo = c_ref[0, 0, :]
        onehot = (combo[:, None]
                  == lax.broadcasted_iota(jnp.int32, (_EBLK, 128), 1)
                  ).astype(jnp.float32)
        gate = jnp.dot(onehot, g_ref[...], preferred_element_type=jnp.float32)
        exb = jnp.dot(ex, s4_ref[...], preferred_element_type=jnp.float32)
        opw_ref[...] = hs * gate * exb
        oex_ref[...] = exb

    return pl.pallas_call(
        body,
        grid=(nblk,),
        in_specs=[
            pl.BlockSpec((_EBLK, _C), lambda i: (i, 0)),
            pl.BlockSpec((_EBLK, _C), lambda i: (i, 0)),
            pl.BlockSpec((1, 1, _EBLK), lambda i: (i, 0, 0)),
            pl.BlockSpec((128, _C), lambda i: (0, 0)),
            pl.BlockSpec((_C, _H), lambda i: (0, 0)),
            pl.BlockSpec((_H, _C), lambda i: (0, 0)),
        ],
        out_specs=[
            pl.BlockSpec((_EBLK, _C), lambda i: (i, 0)),
            pl.BlockSpec((_EBLK, _C), lambda i: (i, 0)),
        ],
        out_shape=[
            jax.ShapeDtypeStruct((E, _C), jnp.float32),
            jax.ShapeDtypeStruct((E, _C), jnp.float32),
        ],
    )(hs_e, hd_e, combo3, G, A, S4)


def _final(agg_p, s_p, W_out, b_out, block_rows=1000):
    n = agg_p.shape[1]

    def body(a_ref, s_ref, w_ref, b_ref, o_ref):
        agg = a_ref[0] + a_ref[1]
        srep = s_ref[0] + s_ref[1]
        norm = agg / (srep + 1e-16)
        o_ref[...] = jnp.dot(norm, w_ref[...],
                             preferred_element_type=jnp.float32) + b_ref[...]

    return pl.pallas_call(
        body,
        grid=(n // block_rows,),
        in_specs=[
            pl.BlockSpec((2, block_rows, _C), lambda i: (0, i, 0)),
            pl.BlockSpec((2, block_rows, _C), lambda i: (0, i, 0)),
            pl.BlockSpec((_C, _C), lambda i: (0, 0)),
            pl.BlockSpec((1, _C), lambda i: (0, 0)),
        ],
        out_specs=pl.BlockSpec((block_rows, _C), lambda i: (i, 0)),
        out_shape=jax.ShapeDtypeStruct((n, _C), jnp.float32),
    )(agg_p, s_p, W_out, b_out.reshape(1, -1))


def _conv(x_src, x_dst, ei, ea, p, ek, emb_t, emb_s, n_dst):
    src, dst = ei[0], ei[1]
    E = ei.shape[1]

    Hs = _mm_bias(x_src, p["W_src_" + ek], p["b_src_" + ek], block_rows=1000)
    Hd = _mm_bias(x_dst, p["W_dst_" + ek], p["b_dst_" + ek], block_rows=1000)

    n_t, n_s = emb_t.shape[0], emb_s.shape[0]
    cat = jnp.concatenate(
        [jnp.repeat(emb_t, n_s, axis=0), jnp.tile(emb_s, (n_t, 1))], axis=1)
    G = _mm_bias(cat, p["W_gate_" + ek], p["b_gate_" + ek], sigmoid=True)

    attn = p["attn_" + ek].reshape(-1)
    head_of = jnp.arange(_C) // _D
    A = jnp.zeros((_C, _H), jnp.float32).at[jnp.arange(_C), head_of].set(attn)
    S4 = (head_of[None, :] == jnp.arange(_H)[:, None]).astype(jnp.float32)

    combo3 = (ea[:, 0] * n_s + ea[:, 1]).astype(jnp.int32).reshape(
        E // _EBLK, 1, _EBLK)

    hs_e, hd_e = _sc_gather(Hs, Hd, src, dst)
    pw, exb = _tc_middle(hs_e, hd_e, combo3, G, A, S4)
    zc = jnp.zeros((n_dst, _C), jnp.float32)
    agg_p = _sc_scatter(pw, dst, zc)
    s_p = _sc_scatter(exb, dst, zc)
    return agg_p, s_p


def kernel(x_chemical, x_gene, edge_index_cg, edge_index_gc, edge_attr_cg,
           edge_attr_gc, params):
    p = params
    emb_t, emb_s = p["emb_action_type"], p["emb_action_subject"]
    agg_g, s_g = _conv(x_chemical, x_gene, edge_index_cg, edge_attr_cg,
                       p, "cg", emb_t, emb_s, x_gene.shape[0])
    agg_c, s_c = _conv(x_gene, x_chemical, edge_index_gc, edge_attr_gc,
                       p, "gc", emb_t, emb_s, x_chemical.shape[0])
    out_chemical = _final(agg_c, s_c, p["W_out_chemical"], p["b_out_chemical"])
    out_gene = _final(agg_g, s_g, p["W_out_gene"], p["b_out_gene"])
    return (out_chemical, out_gene)


# speedup vs baseline: 8.9430x; 1.2749x over previous
import functools

import jax
import jax.numpy as jnp
from jax import lax
from jax.experimental import pallas as pl
from jax.experimental.pallas import tpu as pltpu
from jax.experimental.pallas import tpu_sc as plsc

_C = 128
_H = 4
_D = 32
_NTILE = 32
_CHUNK = 128
_EBLK = 1280


def _mm_bias(x, W, b, sigmoid=False, block_rows=None):
    n = x.shape[0]
    if block_rows is None:
        block_rows = n

    def body(x_ref, w_ref, b_ref, o_ref):
        acc = jnp.dot(x_ref[...], w_ref[...], preferred_element_type=jnp.float32)
        acc = acc + b_ref[...]
        if sigmoid:
            acc = jax.nn.sigmoid(acc)
        o_ref[...] = acc

    return pl.pallas_call(
        body,
        grid=(n // block_rows,),
        in_specs=[
            pl.BlockSpec((block_rows, x.shape[1]), lambda i: (i, 0)),
            pl.BlockSpec((x.shape[1], W.shape[1]), lambda i: (0, 0)),
            pl.BlockSpec((1, W.shape[1]), lambda i: (0, 0)),
        ],
        out_specs=pl.BlockSpec((block_rows, W.shape[1]), lambda i: (i, 0)),
        out_shape=jax.ShapeDtypeStruct((n, W.shape[1]), jnp.float32),
    )(x, W, b.reshape(1, -1))


def _sc_gather(Hs, Hd, src, dst):
    E = src.shape[0]
    per = E // _NTILE
    nfull = per // _CHUNK
    tail = per - nfull * _CHUNK
    mesh = plsc.VectorSubcoreMesh(core_axis_name="c", subcore_axis_name="s")

    assert nfull % 2 == 0
    scratch = []
    for _ in range(2):
        scratch += [
            pltpu.VMEM((_CHUNK,), jnp.int32),
            pltpu.VMEM((_CHUNK,), jnp.int32),
            pltpu.VMEM((_CHUNK, _C), jnp.float32),
            pltpu.VMEM((_CHUNK, _C), jnp.float32),
            pltpu.SemaphoreType.DMA,
            pltpu.SemaphoreType.DMA,
        ]
    scratch += [
        pltpu.VMEM((max(tail, 8),), jnp.int32),
        pltpu.VMEM((max(tail, 8),), jnp.int32),
        pltpu.VMEM((max(tail, 8), _C), jnp.float32),
        pltpu.VMEM((max(tail, 8), _C), jnp.float32),
    ]

    @functools.partial(
        pl.kernel,
        out_type=[jax.ShapeDtypeStruct((E, _C), jnp.float32),
                  jax.ShapeDtypeStruct((E, _C), jnp.float32)],
        mesh=mesh,
        scratch_types=scratch,
    )
    def k(hs_hbm, hd_hbm, src_hbm, dst_hbm, os_hbm, od_hbm,
          isv0, idv0, rs0, rd0, sa0, sb0,
          isv1, idv1, rs1, rd1, sa1, sb1,
          isvt, idvt, rst, rdt):
        wid = lax.axis_index("s") * 2 + lax.axis_index("c")
        base = wid * per
        isv = (isv0, isv1)
        idv = (idv0, idv1)
        rs = (rs0, rs1)
        rd = (rd0, rd1)
        sa = (sa0, sa1)
        sb = (sb0, sb1)

        def fire(i, b):
            off = base + i * _CHUNK
            pltpu.sync_copy(src_hbm.at[pl.ds(off, _CHUNK)], isv[b])
            pltpu.sync_copy(dst_hbm.at[pl.ds(off, _CHUNK)], idv[b])
            pltpu.async_copy(hs_hbm.at[isv[b]], rs[b], sa[b])
            pltpu.async_copy(hd_hbm.at[idv[b]], rd[b], sb[b])

        def finish(i, b):
            pltpu.make_async_copy(hs_hbm.at[pl.ds(0, _CHUNK)], rs[b], sa[b]).wait()
            pltpu.make_async_copy(hd_hbm.at[pl.ds(0, _CHUNK)], rd[b], sb[b]).wait()
            off = base + i * _CHUNK
            pltpu.sync_copy(rs[b], os_hbm.at[pl.ds(off, _CHUNK)])
            pltpu.sync_copy(rd[b], od_hbm.at[pl.ds(off, _CHUNK)])

        fire(0, 0)

        @pl.loop(0, nfull // 2)
        def _(j):
            fire(2 * j + 1, 1)
            finish(2 * j, 0)

            @pl.when(j < nfull // 2 - 1)
            def _():
                fire(2 * j + 2, 0)

            finish(2 * j + 1, 1)

        if tail:
            off = base + nfull * _CHUNK
            pltpu.sync_copy(src_hbm.at[pl.ds(off, tail)], isvt)
            pltpu.sync_copy(dst_hbm.at[pl.ds(off, tail)], idvt)
            c1 = pltpu.async_copy(hs_hbm.at[isvt], rst, sa0)
            c2 = pltpu.async_copy(hd_hbm.at[idvt], rdt, sb0)
            c1.wait()
            c2.wait()
            pltpu.sync_copy(rst, os_hbm.at[pl.ds(off, tail)])
            pltpu.sync_copy(rdt, od_hbm.at[pl.ds(off, tail)])

    return k(Hs, Hd, src, dst)


def _sc_scatter(pw, dst, zc):
    E = pw.shape[0]
    per = E // _NTILE
    nfull = per // _CHUNK
    tail = per - nfull * _CHUNK
    N = zc.shape[0]
    stripe = (N // 16) & ~7
    rem = N - stripe * 16
    mesh = plsc.VectorSubcoreMesh(core_axis_name="c", subcore_axis_name="s")

    @functools.partial(
        pl.kernel,
        out_type=jax.ShapeDtypeStruct((2, N, _C), jnp.float32),
        mesh=mesh,
        scratch_types=[
            pltpu.VMEM((_CHUNK,), jnp.int32),
            pltpu.VMEM((_CHUNK,), jnp.int32),
            pltpu.VMEM((max(tail, 8),), jnp.int32),
            pltpu.VMEM((_CHUNK, _C), jnp.float32),
            pltpu.VMEM((_CHUNK, _C), jnp.float32),
            pltpu.VMEM((max(tail, 8), _C), jnp.float32),
            pltpu.VMEM((16, _C), jnp.float32),
            pltpu.SemaphoreType.DMA,
            pltpu.SemaphoreType.DMA,
            pltpu.VMEM_SHARED((N, _C), jnp.float32),
        ],
    )
    def k(pw_hbm, dst_hbm, zc_hbm, oa_hbm,
          idx0, idx1, idxt, pv0, pv1, pvt, stg, sl0, sl1, aggsh):
        cid = lax.axis_index("c")
        sid = lax.axis_index("s")
        wid = sid * 2 + cid
        base = wid * per
        r0 = sid * stripe
        nz = stripe // 16

        pltpu.sync_copy(zc_hbm.at[pl.ds(0, 16)], stg)

        @pl.loop(0, nz)
        def _(i):
            pltpu.sync_copy(stg, aggsh.at[pl.ds(r0 + i * 16, 16)])

        @pl.when(sid == 15)
        def _():
            @pl.loop(0, rem // 16)
            def _(i):
                pltpu.sync_copy(stg, aggsh.at[pl.ds(stripe * 16 + i * 16, 16)])

        plsc.subcore_barrier()

        idx = (idx0, idx1)
        pv = (pv0, pv1)
        sl = (sl0, sl1)

        def fire(i, b):
            off = base + i * _CHUNK
            pltpu.async_copy(dst_hbm.at[pl.ds(off, _CHUNK)], idx[b], sl[b])
            pltpu.async_copy(pw_hbm.at[pl.ds(off, _CHUNK)], pv[b], sl[b])

        def drain(i, b):
            pltpu.make_async_copy(dst_hbm.at[pl.ds(base, _CHUNK)], idx[b], sl[b]).wait()
            pltpu.make_async_copy(pw_hbm.at[pl.ds(base, _CHUNK)], pv[b], sl[b]).wait()
            pltpu.sync_copy(pv[b], aggsh.at[idx[b]], add=True)

        fire(0, 0)

        @pl.loop(0, nfull // 2)
        def _(j):
            fire(2 * j + 1, 1)
            drain(2 * j, 0)

            @pl.when(j < nfull // 2 - 1)
            def _():
                fire(2 * j + 2, 0)

            drain(2 * j + 1, 1)

        if tail:
            off = base + nfull * _CHUNK
            pltpu.sync_copy(dst_hbm.at[pl.ds(off, tail)], idxt)
            pltpu.sync_copy(pw_hbm.at[pl.ds(off, tail)], pvt)
            pltpu.sync_copy(pvt, aggsh.at[idxt], add=True)

        plsc.subcore_barrier()

        @pl.loop(0, nz)
        def _(i):
            pltpu.sync_copy(aggsh.at[pl.ds(r0 + i * 16, 16)], stg)
            pltpu.sync_copy(stg, oa_hbm.at[cid].at[pl.ds(r0 + i * 16, 16)])

        @pl.when(sid == 15)
        def _():
            @pl.loop(0, rem // 16)
            def _(i):
                pltpu.sync_copy(aggsh.at[pl.ds(stripe * 16 + i * 16, 16)], stg)
                pltpu.sync_copy(stg, oa_hbm.at[cid].at[pl.ds(stripe * 16 + i * 16, 16)])

    return k(pw, dst, zc)


def _tc_middle(hs_e, hd_e, combo3, G, A, S4):
    E = hs_e.shape[0]
    nblk = E // _EBLK

    def body(hs_ref, hd_ref, c_ref, g_ref, a_ref, s4_ref, opw_ref, oex_ref):
        hs = hs_ref[...]
        t = hs + hd_ref[...]
        t = jnp.where(t >= 0, t, 0.2 * t)
        logits = jnp.dot(t, a_ref[...], preferred_element_type=jnp.float32)
        ex = jnp.exp(jnp.minimum(logits, 60.0))
        combo = c_ref[0, 0, :]
        onehot = (combo[:, None]
                  == lax.broadcasted_iota(jnp.int32, (_EBLK, 128), 1)
                  ).astype(jnp.float32)
        gate = jnp.dot(onehot, g_ref[...], preferred_element_type=jnp.float32)
        exb = jnp.dot(ex, s4_ref[...], preferred_element_type=jnp.float32)
        opw_ref[...] = hs * gate * exb
        oex_ref[...] = exb

    return pl.pallas_call(
        body,
        grid=(nblk,),
        in_specs=[
            pl.BlockSpec((_EBLK, _C), lambda i: (i, 0)),
            pl.BlockSpec((_EBLK, _C), lambda i: (i, 0)),
            pl.BlockSpec((1, 1, _EBLK), lambda i: (i, 0, 0)),
            pl.BlockSpec((128, _C), lambda i: (0, 0)),
            pl.BlockSpec((_C, _H), lambda i: (0, 0)),
            pl.BlockSpec((_H, _C), lambda i: (0, 0)),
        ],
        out_specs=[
            pl.BlockSpec((_EBLK, _C), lambda i: (i, 0)),
            pl.BlockSpec((_EBLK, _C), lambda i: (i, 0)),
        ],
        out_shape=[
            jax.ShapeDtypeStruct((E, _C), jnp.float32),
            jax.ShapeDtypeStruct((E, _C), jnp.float32),
        ],
    )(hs_e, hd_e, combo3, G, A, S4)


def _final(agg_p, s_p, W_out, b_out, block_rows=1000):
    n = agg_p.shape[1]

    def body(a_ref, s_ref, w_ref, b_ref, o_ref):
        agg = a_ref[0] + a_ref[1]
        srep = s_ref[0] + s_ref[1]
        norm = agg / (srep + 1e-16)
        o_ref[...] = jnp.dot(norm, w_ref[...],
                             preferred_element_type=jnp.float32) + b_ref[...]

    return pl.pallas_call(
        body,
        grid=(n // block_rows,),
        in_specs=[
            pl.BlockSpec((2, block_rows, _C), lambda i: (0, i, 0)),
            pl.BlockSpec((2, block_rows, _C), lambda i: (0, i, 0)),
            pl.BlockSpec((_C, _C), lambda i: (0, 0)),
            pl.BlockSpec((1, _C), lambda i: (0, 0)),
        ],
        out_specs=pl.BlockSpec((block_rows, _C), lambda i: (i, 0)),
        out_shape=jax.ShapeDtypeStruct((n, _C), jnp.float32),
    )(agg_p, s_p, W_out, b_out.reshape(1, -1))


def _conv(x_src, x_dst, ei, ea, p, ek, emb_t, emb_s, n_dst):
    src, dst = ei[0], ei[1]
    E = ei.shape[1]

    Hs = _mm_bias(x_src, p["W_src_" + ek], p["b_src_" + ek], block_rows=1000)
    Hd = _mm_bias(x_dst, p["W_dst_" + ek], p["b_dst_" + ek], block_rows=1000)

    n_t, n_s = emb_t.shape[0], emb_s.shape[0]
    cat = jnp.concatenate(
        [jnp.repeat(emb_t, n_s, axis=0), jnp.tile(emb_s, (n_t, 1))], axis=1)
    G = _mm_bias(cat, p["W_gate_" + ek], p["b_gate_" + ek], sigmoid=True)

    attn = p["attn_" + ek].reshape(-1)
    head_of = jnp.arange(_C) // _D
    A = jnp.zeros((_C, _H), jnp.float32).at[jnp.arange(_C), head_of].set(attn)
    S4 = (head_of[None, :] == jnp.arange(_H)[:, None]).astype(jnp.float32)

    combo3 = (ea[:, 0] * n_s + ea[:, 1]).astype(jnp.int32).reshape(
        E // _EBLK, 1, _EBLK)

    hs_e, hd_e = _sc_gather(Hs, Hd, src, dst)
    pw, exb = _tc_middle(hs_e, hd_e, combo3, G, A, S4)
    zc = jnp.zeros((n_dst, _C), jnp.float32)
    agg_p = _sc_scatter(pw, dst, zc)
    s_p = _sc_scatter(exb, dst, zc)
    return agg_p, s_p


def kernel(x_chemical, x_gene, edge_index_cg, edge_index_gc, edge_attr_cg,
           edge_attr_gc, params):
    p = params
    emb_t, emb_s = p["emb_action_type"], p["emb_action_subject"]
    agg_g, s_g = _conv(x_chemical, x_gene, edge_index_cg, edge_attr_cg,
                       p, "cg", emb_t, emb_s, x_gene.shape[0])
    agg_c, s_c = _conv(x_gene, x_chemical, edge_index_gc, edge_attr_gc,
                       p, "gc", emb_t, emb_s, x_chemical.shape[0])
    out_chemical = _final(agg_c, s_c, p["W_out_chemical"], p["b_out_chemical"])
    out_gene = _final(agg_g, s_g, p["W_out_gene"], p["b_out_gene"])
    return (out_chemical, out_gene)
